# Initial kernel scaffold; baseline (speedup 1.0000x reference)
import jax
import jax.numpy as jnp
from jax.experimental import pallas as pl


def kernel(v, batch_idx, mask, count, W1, b1, W2, b2):
    raise NotImplementedError("write your pallas kernel here")


# speedup vs baseline: 1.5074x; 1.5074x over previous
import functools

import jax
import jax.numpy as jnp
from jax import lax
from jax.experimental import pallas as pl
from jax.experimental.pallas import tpu as pltpu
from jax.experimental.pallas import tpu_sc as plsc


def _select_body(maskf_ref, bidx_ref, gidx_ref, total_ref, *, L, K):
    m = maskf_ref[...]
    r = lax.broadcasted_iota(jnp.int32, (L, L), 0)
    c = lax.broadcasted_iota(jnp.int32, (L, L), 1)
    tri = (r <= c).astype(jnp.float32)
    cs = jnp.dot(m, tri, preferred_element_type=jnp.float32)
    pos = lax.broadcasted_iota(jnp.int32, m.shape, 1).astype(jnp.float32)
    cols = []
    for j in range(K):
        sel = m * (cs == float(j + 1)).astype(jnp.float32)
        cols.append(jnp.sum(sel * pos, axis=1, keepdims=True))
    idxf = jnp.concatenate(cols, axis=1)
    gidx_ref[...] = bidx_ref[...] * L + idxf.astype(jnp.int32)
    total_ref[...] = cs[:, L - 1:L]


def _mlp_body(flat_ref, total_ref, cnt_ref, w1t_ref, w1c_ref, b1_ref,
              w2t_ref, b2_ref, out_ref, *, DV):
    flat = flat_ref[...]
    slot = lax.broadcasted_iota(jnp.int32, flat.shape, 1) // DV
    keep = (slot < total_ref[...].astype(jnp.int32)).astype(jnp.float32)
    x = flat * keep
    lc = jnp.log1p(cnt_ref[...])
    h = jnp.dot(x, w1t_ref[...], preferred_element_type=jnp.float32)
    h = h + lc * w1c_ref[...] + b1_ref[...]
    h = 0.5 * h * (1.0 + lax.erf(h * (2.0 ** -0.5)))
    out_ref[...] = jnp.dot(h, w2t_ref[...], preferred_element_type=jnp.float32) + b2_ref[...]


def _make_sc_gather(n_rows, d, chunk=128):
    info = plsc.get_sparse_core_info()
    nw = info.num_cores * info.num_subcores
    rows_per_w = n_rows // nw
    chunks = rows_per_w // chunk
    mesh = plsc.VectorSubcoreMesh(core_axis_name="c", subcore_axis_name="s")

    @functools.partial(
        pl.kernel, mesh=mesh,
        out_type=jax.ShapeDtypeStruct((n_rows, d), jnp.float32),
        compiler_params=pltpu.CompilerParams(use_tc_tiling_on_sc=False),
        scratch_types=[
            pltpu.VMEM((chunks, chunk), jnp.int32),
            pltpu.VMEM((rows_per_w, d), jnp.float32),
            pltpu.SemaphoreType.DMA,
        ],
    )
    def gather(idx_hbm, table_hbm, out_hbm, idx_v, rows_v, sem):
        wid = lax.axis_index("s") * info.num_cores + lax.axis_index("c")
        pltpu.sync_copy(idx_hbm.at[pl.ds(wid * chunks, chunks)], idx_v)
        copies = []
        for c in range(chunks):
            copies.append(pltpu.async_copy(
                table_hbm.at[idx_v.at[c]],
                rows_v.at[pl.ds(c * chunk, chunk)],
                sem))
        for cp in copies:
            cp.wait()
        pltpu.sync_copy(rows_v, out_hbm.at[pl.ds(wid * rows_per_w, rows_per_w)])

    return gather


def kernel(v, batch_idx, mask, count, W1, b1, W2, b2):
    n_batch, L, dv = v.shape
    n_chains = mask.shape[0]
    hidden, d_in = W1.shape
    K = (d_in - 1) // dv

    maskf = mask.astype(jnp.float32)
    bidx = batch_idx.astype(jnp.int32).reshape(n_chains, 1)

    blk = 1024
    grid = n_chains // blk
    gidx, total = pl.pallas_call(
        functools.partial(_select_body, L=L, K=K),
        grid=(grid,),
        in_specs=[
            pl.BlockSpec((blk, L), lambda i: (i, 0)),
            pl.BlockSpec((blk, 1), lambda i: (i, 0)),
        ],
        out_specs=[
            pl.BlockSpec((blk, K), lambda i: (i, 0)),
            pl.BlockSpec((blk, 1), lambda i: (i, 0)),
        ],
        out_shape=[
            jax.ShapeDtypeStruct((n_chains, K), jnp.int32),
            jax.ShapeDtypeStruct((n_chains, 1), jnp.float32),
        ],
    )(maskf, bidx)

    table = v.reshape(n_batch * L, dv)
    idx2d = gidx.reshape(n_chains * K // 128, 128)
    rows = _make_sc_gather(n_chains * K, dv)(idx2d, table)
    flat = rows.reshape(n_chains, K * dv)

    w1t = W1[:, :K * dv].T
    w1c = W1[:, K * dv:].T
    w2t = W2.T
    out = pl.pallas_call(
        functools.partial(_mlp_body, DV=dv),
        grid=(grid,),
        in_specs=[
            pl.BlockSpec((blk, K * dv), lambda i: (i, 0)),
            pl.BlockSpec((blk, 1), lambda i: (i, 0)),
            pl.BlockSpec((blk, 1), lambda i: (i, 0)),
            pl.BlockSpec((K * dv, hidden), lambda i: (0, 0)),
            pl.BlockSpec((1, hidden), lambda i: (0, 0)),
            pl.BlockSpec((1, hidden), lambda i: (0, 0)),
            pl.BlockSpec((hidden, dv), lambda i: (0, 0)),
            pl.BlockSpec((1, dv), lambda i: (0, 0)),
        ],
        out_specs=pl.BlockSpec((blk, dv), lambda i: (i, 0)),
        out_shape=jax.ShapeDtypeStruct((n_chains, dv), jnp.float32),
    )(flat, total, count.reshape(n_chains, 1).astype(jnp.float32),
      w1t, w1c, b1.reshape(1, hidden), w2t, b2.reshape(1, dv))
    return out


# speedup vs baseline: 2.6142x; 1.7343x over previous
import functools

import jax
import jax.numpy as jnp
from jax import lax
from jax.experimental import pallas as pl
from jax.experimental.pallas import tpu as pltpu
from jax.experimental.pallas import tpu_sc as plsc


def _transpose_body(vt_ref, out_ref):
    x0 = vt_ref[0]
    x1 = vt_ref[1]
    y0 = jnp.transpose(x0, (1, 0))
    y1 = jnp.transpose(x1, (1, 0))
    out_ref[...] = jnp.concatenate([y0, y1], axis=1)[None]


def _select_body(maskf_ref, bidx_ref, gidx_ref, par_ref, total_ref, *, L, K, NB):
    m = maskf_ref[...]
    r = lax.broadcasted_iota(jnp.int32, (L, L), 0)
    c = lax.broadcasted_iota(jnp.int32, (L, L), 1)
    tri = (r <= c).astype(jnp.float32)
    cs = jnp.dot(m, tri, preferred_element_type=jnp.float32)
    pos = lax.broadcasted_iota(jnp.int32, m.shape, 1).astype(jnp.float32)
    cols = []
    for j in range(K):
        sel = m * (cs == float(j + 1)).astype(jnp.float32)
        cols.append(jnp.sum(sel * pos, axis=1, keepdims=True))
    idxf = jnp.concatenate(cols, axis=1)
    idx = idxf.astype(jnp.int32)
    gidx_ref[...] = (idx // 2) * NB + bidx_ref[...]
    par_ref[...] = (idx % 2).astype(jnp.float32)
    total_ref[...] = cs[:, L - 1:L]


def _mlp_body(flat_ref, par_ref, total_ref, cnt_ref, w1t_ref, w1c_ref, b1_ref,
              w2t_ref, b2_ref, out_ref, *, DV, K):
    x = flat_ref[...]
    w2 = 2 * DV
    col = lax.broadcasted_iota(jnp.int32, x.shape, 1)
    slot = col // w2
    half = ((col % w2) // DV).astype(jnp.float32)
    keep = (slot < total_ref[...].astype(jnp.int32)).astype(jnp.float32)
    e1 = lax.broadcasted_iota(jnp.int32, (K, K * w2), 0)
    e2 = lax.broadcasted_iota(jnp.int32, (K, K * w2), 1) // w2
    eb = (e1 == e2).astype(jnp.float32)
    par = jnp.dot(par_ref[...], eb, preferred_element_type=jnp.float32)
    match = 1.0 - par - half + 2.0 * par * half
    x = x * (match * keep)
    lc = jnp.log1p(cnt_ref[...])
    h = jnp.dot(x, w1t_ref[...], preferred_element_type=jnp.float32)
    h = h + lc * w1c_ref[...] + b1_ref[...]
    h = 0.5 * h * (1.0 + lax.erf(h * (2.0 ** -0.5)))
    out_ref[...] = jnp.dot(h, w2t_ref[...], preferred_element_type=jnp.float32) + b2_ref[...]


def _make_sc_gather(n_rows, d, chunk=128):
    info = plsc.get_sparse_core_info()
    nw = info.num_cores * info.num_subcores
    rows_per_w = n_rows // nw
    chunks = rows_per_w // chunk
    half = chunks // 2
    mesh = plsc.VectorSubcoreMesh(core_axis_name="c", subcore_axis_name="s")

    @functools.partial(
        pl.kernel, mesh=mesh,
        out_type=jax.ShapeDtypeStruct((n_rows, d), jnp.float32),
        compiler_params=pltpu.CompilerParams(use_tc_tiling_on_sc=False),
        scratch_types=[
            pltpu.VMEM((chunks, chunk), jnp.int32),
            pltpu.VMEM((half * chunk, d), jnp.float32),
            pltpu.SemaphoreType.DMA,
        ],
    )
    def gather(idx_hbm, table_hbm, out_hbm, idx_v, rows_v, sem):
        wid = lax.axis_index("s") * info.num_cores + lax.axis_index("c")
        pltpu.sync_copy(idx_hbm.at[pl.ds(wid * chunks, chunks)], idx_v)
        for rnd in range(2):
            copies = []
            for c in range(half):
                copies.append(pltpu.async_copy(
                    table_hbm.at[idx_v.at[rnd * half + c]],
                    rows_v.at[pl.ds(c * chunk, chunk)],
                    sem))
            for cp in copies:
                cp.wait()
            pltpu.sync_copy(
                rows_v,
                out_hbm.at[pl.ds(wid * rows_per_w + rnd * half * chunk, half * chunk)])

    return gather


def kernel(v, batch_idx, mask, count, W1, b1, W2, b2):
    n_batch, L, dv = v.shape
    n_chains = mask.shape[0]
    hidden, d_in = W1.shape
    K = (d_in - 1) // dv
    q = L // 2

    vt = jnp.transpose(v, (1, 2, 0))
    table3 = pl.pallas_call(
        _transpose_body,
        grid=(q,),
        in_specs=[pl.BlockSpec((2, dv, n_batch), lambda i: (i, 0, 0))],
        out_specs=pl.BlockSpec((1, n_batch, 2 * dv), lambda i: (i, 0, 0)),
        out_shape=jax.ShapeDtypeStruct((q, n_batch, 2 * dv), jnp.float32),
    )(vt)
    table = table3.reshape(q * n_batch, 2 * dv)

    maskf = mask.astype(jnp.float32)
    bidx = batch_idx.astype(jnp.int32).reshape(n_chains, 1)
    blk = 1024
    grid = n_chains // blk
    gidx, par, total = pl.pallas_call(
        functools.partial(_select_body, L=L, K=K, NB=n_batch),
        grid=(grid,),
        in_specs=[
            pl.BlockSpec((blk, L), lambda i: (i, 0)),
            pl.BlockSpec((blk, 1), lambda i: (i, 0)),
        ],
        out_specs=[
            pl.BlockSpec((blk, K), lambda i: (i, 0)),
            pl.BlockSpec((blk, K), lambda i: (i, 0)),
            pl.BlockSpec((blk, 1), lambda i: (i, 0)),
        ],
        out_shape=[
            jax.ShapeDtypeStruct((n_chains, K), jnp.int32),
            jax.ShapeDtypeStruct((n_chains, K), jnp.float32),
            jax.ShapeDtypeStruct((n_chains, 1), jnp.float32),
        ],
    )(maskf, bidx)

    idx2d = gidx.reshape(n_chains * K // 128, 128)
    rows = _make_sc_gather(n_chains * K, 2 * dv)(idx2d, table)
    flat = rows.reshape(n_chains, K * 2 * dv)

    w1t = W1[:, :K * dv].T
    w1t_dup = jnp.broadcast_to(
        w1t.reshape(K, 1, dv, hidden), (K, 2, dv, hidden)).reshape(K * 2 * dv, hidden)
    w1c = W1[:, K * dv:].T
    w2t = W2.T
    out = pl.pallas_call(
        functools.partial(_mlp_body, DV=dv, K=K),
        grid=(grid,),
        in_specs=[
            pl.BlockSpec((blk, K * 2 * dv), lambda i: (i, 0)),
            pl.BlockSpec((blk, K), lambda i: (i, 0)),
            pl.BlockSpec((blk, 1), lambda i: (i, 0)),
            pl.BlockSpec((blk, 1), lambda i: (i, 0)),
            pl.BlockSpec((K * 2 * dv, hidden), lambda i: (0, 0)),
            pl.BlockSpec((1, hidden), lambda i: (0, 0)),
            pl.BlockSpec((1, hidden), lambda i: (0, 0)),
            pl.BlockSpec((hidden, dv), lambda i: (0, 0)),
            pl.BlockSpec((1, dv), lambda i: (0, 0)),
        ],
        out_specs=pl.BlockSpec((blk, dv), lambda i: (i, 0)),
        out_shape=jax.ShapeDtypeStruct((n_chains, dv), jnp.float32),
    )(flat, par, total, count.reshape(n_chains, 1).astype(jnp.float32),
      w1t_dup, w1c, b1.reshape(1, hidden), w2t, b2.reshape(1, dv))
    return out


# speedup vs baseline: 4.6171x; 1.7661x over previous
import functools

import jax
import jax.numpy as jnp
from jax import lax
from jax.experimental import pallas as pl
from jax.experimental.pallas import tpu as pltpu
from jax.experimental.pallas import tpu_sc as plsc

T_FAST = 64


def _transpose_body(vt_ref, out_ref):
    x0 = vt_ref[0]
    x1 = vt_ref[1]
    y0 = jnp.transpose(x0, (1, 0))
    y1 = jnp.transpose(x1, (1, 0))
    out_ref[...] = jnp.concatenate([y0, y1], axis=1)[None]


def _select_body(maskf_ref, bidx_ref, gidx_ref, par_ref, rare_ref, total_ref,
                 *, L, K, NB, DV):
    m = maskf_ref[...]
    r = lax.broadcasted_iota(jnp.int32, (L, L), 0)
    c = lax.broadcasted_iota(jnp.int32, (L, L), 1)
    tri = (r <= c).astype(jnp.float32)
    cs = jnp.dot(m, tri, preferred_element_type=jnp.float32)
    pos = lax.broadcasted_iota(jnp.int32, m.shape, 1).astype(jnp.float32)
    cols = []
    pick = []
    for j in range(K):
        sel = m * (cs == float(j + 1)).astype(jnp.float32)
        cols.append(jnp.sum(sel * pos, axis=1, keepdims=True))
        pick.append(jnp.sum(sel, axis=1, keepdims=True))
    idxf = jnp.concatenate(cols, axis=1)
    picked = jnp.concatenate(pick, axis=1)
    idx = idxf.astype(jnp.int32)
    bidx = bidx_ref[...]
    fast = idx < T_FAST
    gidx_ref[...] = jnp.where(fast, (idx // 2) * NB + bidx, 0)
    par_ref[...] = (idx % 2).astype(jnp.float32)
    rare = (picked > 0.5) & jnp.logical_not(fast)
    rare_ref[...] = jnp.where(rare, idx * (DV * NB) + bidx, -1)
    total_ref[...] = cs[:, L - 1:L]


def _mlp_body(flat_ref, par_ref, total_ref, cnt_ref, w1t_ref, w1c_ref, b1_ref,
              w2t_ref, b2_ref, out_ref, *, DV, K):
    x = flat_ref[...]
    w2 = 2 * DV
    col = lax.broadcasted_iota(jnp.int32, x.shape, 1)
    slot = col // w2
    half = ((col % w2) // DV).astype(jnp.float32)
    keep = (slot < total_ref[...].astype(jnp.int32)).astype(jnp.float32)
    e1 = lax.broadcasted_iota(jnp.int32, (K, K * w2), 0)
    e2 = lax.broadcasted_iota(jnp.int32, (K, K * w2), 1) // w2
    eb = (e1 == e2).astype(jnp.float32)
    par = jnp.dot(par_ref[...], eb, preferred_element_type=jnp.float32)
    match = 1.0 - par - half + 2.0 * par * half
    x = x * (match * keep)
    lc = jnp.log1p(cnt_ref[...])
    h = jnp.dot(x, w1t_ref[...], preferred_element_type=jnp.float32)
    h = h + lc * w1c_ref[...] + b1_ref[...]
    h = 0.5 * h * (1.0 + lax.erf(h * (2.0 ** -0.5)))
    out_ref[...] = jnp.dot(h, w2t_ref[...], preferred_element_type=jnp.float32) + b2_ref[...]


def _make_sc_gather(n_rows, d, dv, nb):
    info = plsc.get_sparse_core_info()
    nw = info.num_cores * info.num_subcores
    rows_per_w = n_rows // nw
    chunk = 128
    chunks = rows_per_w // chunk
    half = chunks // 2
    mesh = plsc.VectorSubcoreMesh(core_axis_name="c", subcore_axis_name="s")

    @functools.partial(
        pl.kernel, mesh=mesh,
        out_type=jax.ShapeDtypeStruct((n_rows, d), jnp.float32),
        compiler_params=pltpu.CompilerParams(
            use_tc_tiling_on_sc=True, needs_layout_passes=False),
        scratch_types=[
            pltpu.VMEM((chunks, chunk), jnp.int32),
            pltpu.VMEM((half * chunk, d), jnp.float32),
            pltpu.VMEM((chunks, chunk), jnp.int32),
            pltpu.VMEM((d, d), jnp.float32),
            pltpu.SemaphoreType.DMA,
        ],
    )
    def gather(idx_hbm, rare_hbm, table_hbm, vt2_hbm, out_hbm,
               idx_v, rows_v, rare_v, tile_v, sem):
        wid = lax.axis_index("s") * info.num_cores + lax.axis_index("c")
        pltpu.sync_copy(idx_hbm.at[pl.ds(wid * chunks, chunks)], idx_v)
        pltpu.sync_copy(rare_hbm.at[pl.ds(wid * chunks, chunks)], rare_v)
        lanes = lax.iota(jnp.int32, 16)

        def patch_slot(s_local, val):
            @pl.when(val >= 0)
            def _():
                l = val // (dv * nb)
                b = val - l * (dv * nb)
                q = l // 2
                colbase = (b // d) * d
                c = b - colbase
                pltpu.sync_copy(
                    vt2_hbm.at[pl.ds(q * d, d), pl.ds(colbase, d)], tile_v)
                for r8 in range(d // 16):
                    vals = plsc.load_gather(
                        tile_v, [lanes + 16 * r8, jnp.zeros((16,), jnp.int32) + c])
                    plsc.store_scatter(
                        rows_v,
                        [jnp.zeros((16,), jnp.int32) + s_local, lanes + 16 * r8],
                        vals)

        def patch_round(rnd):
            for r in range(half):
                def g2body(g2, carry, r=r):
                    gv = rare_v[rnd * half + r, pl.ds(g2 * 16, 16)]

                    @pl.when(jnp.max(gv) >= 0)
                    def _():
                        for i in range(16):
                            patch_slot(r * chunk + g2 * 16 + i, gv[i])
                    return carry
                lax.fori_loop(0, chunk // 16, g2body, 0)

        for rnd in range(2):
            copies = []
            for c in range(half):
                copies.append(pltpu.async_copy(
                    table_hbm.at[idx_v.at[rnd * half + c]],
                    rows_v.at[pl.ds(c * chunk, chunk)],
                    sem))
            for cp in copies:
                cp.wait()
            patch_round(rnd)
            pltpu.sync_copy(
                rows_v,
                out_hbm.at[pl.ds(wid * rows_per_w + rnd * half * chunk, half * chunk)])

    return gather


def kernel(v, batch_idx, mask, count, W1, b1, W2, b2):
    n_batch, L, dv = v.shape
    n_chains = mask.shape[0]
    hidden, d_in = W1.shape
    K = (d_in - 1) // dv
    qf = T_FAST // 2

    vt = jnp.transpose(v, (1, 2, 0))
    table3 = pl.pallas_call(
        _transpose_body,
        grid=(qf,),
        in_specs=[pl.BlockSpec((2, dv, n_batch), lambda i: (i, 0, 0))],
        out_specs=pl.BlockSpec((1, n_batch, 2 * dv), lambda i: (i, 0, 0)),
        out_shape=jax.ShapeDtypeStruct((qf, n_batch, 2 * dv), jnp.float32),
    )(vt)
    table = table3.reshape(qf * n_batch, 2 * dv)
    vt2 = vt.reshape(L * dv, n_batch)

    maskf = mask.astype(jnp.float32)
    bidx = batch_idx.astype(jnp.int32).reshape(n_chains, 1)
    blk = 1024
    grid = n_chains // blk
    gidx, par, rare, total = pl.pallas_call(
        functools.partial(_select_body, L=L, K=K, NB=n_batch, DV=dv),
        grid=(grid,),
        in_specs=[
            pl.BlockSpec((blk, L), lambda i: (i, 0)),
            pl.BlockSpec((blk, 1), lambda i: (i, 0)),
        ],
        out_specs=[
            pl.BlockSpec((blk, K), lambda i: (i, 0)),
            pl.BlockSpec((blk, K), lambda i: (i, 0)),
            pl.BlockSpec((blk, K), lambda i: (i, 0)),
            pl.BlockSpec((blk, 1), lambda i: (i, 0)),
        ],
        out_shape=[
            jax.ShapeDtypeStruct((n_chains, K), jnp.int32),
            jax.ShapeDtypeStruct((n_chains, K), jnp.float32),
            jax.ShapeDtypeStruct((n_chains, K), jnp.int32),
            jax.ShapeDtypeStruct((n_chains, 1), jnp.float32),
        ],
    )(maskf, bidx)

    n_rows = n_chains * K
    nw = 32
    idx2d = gidx.reshape(n_rows // 128, 128)
    rare2d = rare.reshape(n_rows // 128, 128)
    rows = _make_sc_gather(n_rows, 2 * dv, dv, n_batch)(idx2d, rare2d, table, vt2)
    flat = rows.reshape(n_chains, K * 2 * dv)

    w1t = W1[:, :K * dv].T
    w1t_dup = jnp.broadcast_to(
        w1t.reshape(K, 1, dv, hidden), (K, 2, dv, hidden)).reshape(K * 2 * dv, hidden)
    w1c = W1[:, K * dv:].T
    w2t = W2.T
    out = pl.pallas_call(
        functools.partial(_mlp_body, DV=dv, K=K),
        grid=(grid,),
        in_specs=[
            pl.BlockSpec((blk, K * 2 * dv), lambda i: (i, 0)),
            pl.BlockSpec((blk, K), lambda i: (i, 0)),
            pl.BlockSpec((blk, 1), lambda i: (i, 0)),
            pl.BlockSpec((blk, 1), lambda i: (i, 0)),
            pl.BlockSpec((K * 2 * dv, hidden), lambda i: (0, 0)),
            pl.BlockSpec((1, hidden), lambda i: (0, 0)),
            pl.BlockSpec((1, hidden), lambda i: (0, 0)),
            pl.BlockSpec((hidden, dv), lambda i: (0, 0)),
            pl.BlockSpec((1, dv), lambda i: (0, 0)),
        ],
        out_specs=pl.BlockSpec((blk, dv), lambda i: (i, 0)),
        out_shape=jax.ShapeDtypeStruct((n_chains, dv), jnp.float32),
    )(flat, par, total, count.reshape(n_chains, 1).astype(jnp.float32),
      w1t_dup, w1c, b1.reshape(1, hidden), w2t, b2.reshape(1, dv))
    return out


# speedup vs baseline: 5.0752x; 1.0992x over previous
import functools

import jax
import jax.numpy as jnp
from jax import lax
from jax.experimental import pallas as pl
from jax.experimental.pallas import tpu as pltpu
from jax.experimental.pallas import tpu_sc as plsc

T_FAST = 64


def _transpose_body(vt_ref, out_ref):
    x0 = vt_ref[0]
    x1 = vt_ref[1]
    dv = x0.shape[0]
    eye = (lax.broadcasted_iota(jnp.int32, (dv, dv), 0) ==
           lax.broadcasted_iota(jnp.int32, (dv, dv), 1)).astype(jnp.float32)
    y0 = lax.dot_general(x0, eye, (((0,), (0,)), ((), ())),
                         preferred_element_type=jnp.float32)
    y1 = jnp.transpose(x1, (1, 0))
    out_ref[...] = jnp.concatenate([y0, y1], axis=1)[None]


def _select_body(maskf_ref, bidx_ref, gidx_ref, par_ref, rare_ref, total_ref,
                 *, L, K, NB, DV):
    m = maskf_ref[...]
    r = lax.broadcasted_iota(jnp.int32, (L, L), 0)
    c = lax.broadcasted_iota(jnp.int32, (L, L), 1)
    tri = (r <= c).astype(jnp.float32)
    cs = jnp.dot(m, tri, preferred_element_type=jnp.float32)
    pos = lax.broadcasted_iota(jnp.int32, m.shape, 1).astype(jnp.float32)
    cols = []
    pick = []
    for j in range(K):
        sel = m * (cs == float(j + 1)).astype(jnp.float32)
        cols.append(jnp.sum(sel * pos, axis=1, keepdims=True))
        pick.append(jnp.sum(sel, axis=1, keepdims=True))
    idxf = jnp.concatenate(cols, axis=1)
    picked = jnp.concatenate(pick, axis=1)
    idx = idxf.astype(jnp.int32)
    bidx = bidx_ref[...]
    fast = idx < T_FAST
    gidx_ref[...] = jnp.transpose(jnp.where(fast, (idx // 2) * NB + bidx, 0), (1, 0))
    par_ref[...] = (idx % 2).astype(jnp.float32)
    rare = (picked > 0.5) & jnp.logical_not(fast)
    rare_ref[...] = jnp.transpose(jnp.where(rare, idx * (DV * NB) + bidx, -1), (1, 0))
    total_ref[...] = cs[:, L - 1:L]


def _mlp_body(rows_ref, par_ref, total_ref, cnt_ref, w1t_ref, w1c_ref, b1_ref,
              w2t_ref, b2_ref, out_ref, *, DV, K):
    w2 = 2 * DV
    halfc = (lax.broadcasted_iota(jnp.int32, (1, w2), 1) >= DV).astype(jnp.float32)
    total = total_ref[...]
    lc = jnp.log1p(cnt_ref[...])
    h = lc * w1c_ref[...] + b1_ref[...]
    for j in range(K):
        xj = rows_ref[j]
        pj = par_ref[:, j:j + 1]
        keep = (total > float(j)).astype(jnp.float32)
        match = (1.0 - pj) * (1.0 - halfc) + pj * halfc
        xj = xj * (match * keep)
        h = h + jnp.dot(xj, w1t_ref[pl.ds(j * w2, w2), :],
                        preferred_element_type=jnp.float32)
    h = 0.5 * h * (1.0 + lax.erf(h * (2.0 ** -0.5)))
    out_ref[...] = jnp.dot(h, w2t_ref[...], preferred_element_type=jnp.float32) + b2_ref[...]


def _make_sc_gather(n_rows, d, dv, nb):
    info = plsc.get_sparse_core_info()
    nw = info.num_cores * info.num_subcores
    rows_per_w = n_rows // nw
    chunk = 128
    chunks = rows_per_w // chunk
    half = chunks // 2
    mesh = plsc.VectorSubcoreMesh(core_axis_name="c", subcore_axis_name="s")

    @functools.partial(
        pl.kernel, mesh=mesh,
        out_type=jax.ShapeDtypeStruct((n_rows, d), jnp.float32),
        compiler_params=pltpu.CompilerParams(
            use_tc_tiling_on_sc=True, needs_layout_passes=False),
        scratch_types=[
            pltpu.VMEM((chunks, chunk), jnp.int32),
            pltpu.VMEM((half * chunk, d), jnp.float32),
            pltpu.VMEM((chunks, chunk), jnp.int32),
            pltpu.VMEM((d, d), jnp.float32),
            pltpu.SemaphoreType.DMA,
        ],
    )
    def gather(idx_hbm, rare_hbm, table_hbm, vt2_hbm, out_hbm,
               idx_v, rows_v, rare_v, tile_v, sem):
        wid = lax.axis_index("s") * info.num_cores + lax.axis_index("c")
        pltpu.sync_copy(idx_hbm.at[pl.ds(wid * chunks, chunks)], idx_v)
        pltpu.sync_copy(rare_hbm.at[pl.ds(wid * chunks, chunks)], rare_v)
        lanes = lax.iota(jnp.int32, 16)

        def patch_slot(s_local, val):
            @pl.when(val >= 0)
            def _():
                l = val // (dv * nb)
                b = val - l * (dv * nb)
                q = l // 2
                colbase = (b // d) * d
                c = b - colbase
                pltpu.sync_copy(
                    vt2_hbm.at[pl.ds(q * d, d), pl.ds(colbase, d)], tile_v)
                for r8 in range(d // 16):
                    vals = plsc.load_gather(
                        tile_v, [lanes + 16 * r8, jnp.zeros((16,), jnp.int32) + c])
                    plsc.store_scatter(
                        rows_v,
                        [jnp.zeros((16,), jnp.int32) + s_local, lanes + 16 * r8],
                        vals)

        def patch_round(rnd):
            for r in range(half):
                def g2body(g2, carry, r=r):
                    gv = rare_v[rnd * half + r, pl.ds(g2 * 16, 16)]

                    @pl.when(jnp.max(gv) >= 0)
                    def _():
                        for i in range(16):
                            patch_slot(r * chunk + g2 * 16 + i, gv[i])
                    return carry
                lax.fori_loop(0, chunk // 16, g2body, 0)

        for rnd in range(2):
            copies = []
            for c in range(half):
                copies.append(pltpu.async_copy(
                    table_hbm.at[idx_v.at[rnd * half + c]],
                    rows_v.at[pl.ds(c * chunk, chunk)],
                    sem))
            for cp in copies:
                cp.wait()
            patch_round(rnd)
            pltpu.sync_copy(
                rows_v,
                out_hbm.at[pl.ds(wid * rows_per_w + rnd * half * chunk, half * chunk)])

    return gather


def kernel(v, batch_idx, mask, count, W1, b1, W2, b2):
    n_batch, L, dv = v.shape
    n_chains = mask.shape[0]
    hidden, d_in = W1.shape
    K = (d_in - 1) // dv
    qf = T_FAST // 2

    vt = jnp.transpose(v, (1, 2, 0))
    table3 = pl.pallas_call(
        _transpose_body,
        grid=(qf,),
        in_specs=[pl.BlockSpec((2, dv, n_batch), lambda i: (i, 0, 0))],
        out_specs=pl.BlockSpec((1, n_batch, 2 * dv), lambda i: (i, 0, 0)),
        out_shape=jax.ShapeDtypeStruct((qf, n_batch, 2 * dv), jnp.float32),
    )(vt)
    table = table3.reshape(qf * n_batch, 2 * dv)
    vt2 = vt.reshape(L * dv, n_batch)

    maskf = mask.astype(jnp.float32)
    bidx = batch_idx.astype(jnp.int32).reshape(n_chains, 1)
    blk = 1024
    grid = n_chains // blk
    gidx, par, rare, total = pl.pallas_call(
        functools.partial(_select_body, L=L, K=K, NB=n_batch, DV=dv),
        grid=(grid,),
        in_specs=[
            pl.BlockSpec((blk, L), lambda i: (i, 0)),
            pl.BlockSpec((blk, 1), lambda i: (i, 0)),
        ],
        out_specs=[
            pl.BlockSpec((K, blk), lambda i: (0, i)),
            pl.BlockSpec((blk, K), lambda i: (i, 0)),
            pl.BlockSpec((K, blk), lambda i: (0, i)),
            pl.BlockSpec((blk, 1), lambda i: (i, 0)),
        ],
        out_shape=[
            jax.ShapeDtypeStruct((K, n_chains), jnp.int32),
            jax.ShapeDtypeStruct((n_chains, K), jnp.float32),
            jax.ShapeDtypeStruct((K, n_chains), jnp.int32),
            jax.ShapeDtypeStruct((n_chains, 1), jnp.float32),
        ],
    )(maskf, bidx)

    n_rows = n_chains * K
    nw = 32
    idx2d = gidx.reshape(n_rows // 128, 128)
    rare2d = rare.reshape(n_rows // 128, 128)
    rows = _make_sc_gather(n_rows, 2 * dv, dv, n_batch)(idx2d, rare2d, table, vt2)
    rows2 = rows.reshape(K, n_chains, 2 * dv)

    w1t = W1[:, :K * dv].T
    w1t_dup = jnp.broadcast_to(
        w1t.reshape(K, 1, dv, hidden), (K, 2, dv, hidden)).reshape(K * 2 * dv, hidden)
    w1c = W1[:, K * dv:].T
    w2t = W2.T
    out = pl.pallas_call(
        functools.partial(_mlp_body, DV=dv, K=K),
        grid=(grid,),
        in_specs=[
            pl.BlockSpec((K, blk, 2 * dv), lambda i: (0, i, 0)),
            pl.BlockSpec((blk, K), lambda i: (i, 0)),
            pl.BlockSpec((blk, 1), lambda i: (i, 0)),
            pl.BlockSpec((blk, 1), lambda i: (i, 0)),
            pl.BlockSpec((K * 2 * dv, hidden), lambda i: (0, 0)),
            pl.BlockSpec((1, hidden), lambda i: (0, 0)),
            pl.BlockSpec((1, hidden), lambda i: (0, 0)),
            pl.BlockSpec((hidden, dv), lambda i: (0, 0)),
            pl.BlockSpec((1, dv), lambda i: (0, 0)),
        ],
        out_specs=pl.BlockSpec((blk, dv), lambda i: (i, 0)),
        out_shape=jax.ShapeDtypeStruct((n_chains, dv), jnp.float32),
    )(rows2, par, total, count.reshape(n_chains, 1).astype(jnp.float32),
      w1t_dup, w1c, b1.reshape(1, hidden), w2t, b2.reshape(1, dv))
    return out


# speedup vs baseline: 5.4015x; 1.0643x over previous
import functools

import jax
import jax.numpy as jnp
from jax import lax
from jax.experimental import pallas as pl
from jax.experimental.pallas import tpu as pltpu
from jax.experimental.pallas import tpu_sc as plsc

T_FAST = 64


def _transpose_body(vt_ref, out_ref):
    dv = vt_ref.shape[1]
    eye = (lax.broadcasted_iota(jnp.int32, (dv, dv), 0) ==
           lax.broadcasted_iota(jnp.int32, (dv, dv), 1)).astype(jnp.float32)
    for q in range(vt_ref.shape[0] // 2):
        x0 = vt_ref[2 * q]
        x1 = vt_ref[2 * q + 1]
        y0 = lax.dot_general(x0, eye, (((0,), (0,)), ((), ())),
                             preferred_element_type=jnp.float32)
        y1 = jnp.transpose(x1, (1, 0))
        out_ref[q] = jnp.concatenate([y0, y1], axis=1)


def _select_body(maskf_ref, bidx_ref, gidx_ref, par_ref, rare_ref, total_ref,
                 *, L, K, NB, DV):
    m = maskf_ref[...]
    r = lax.broadcasted_iota(jnp.int32, (L, L), 0)
    c = lax.broadcasted_iota(jnp.int32, (L, L), 1)
    tri = (r <= c).astype(jnp.float32)
    cs = jnp.dot(m, tri, preferred_element_type=jnp.float32)
    pos = lax.broadcasted_iota(jnp.int32, m.shape, 1).astype(jnp.float32)
    cols = []
    pick = []
    for j in range(K):
        sel = m * (cs == float(j + 1)).astype(jnp.float32)
        cols.append(jnp.sum(sel * pos, axis=1, keepdims=True))
        pick.append(jnp.sum(sel, axis=1, keepdims=True))
    idxf = jnp.concatenate(cols, axis=1)
    picked = jnp.concatenate(pick, axis=1)
    idx = idxf.astype(jnp.int32)
    bidx = bidx_ref[...]
    fast = idx < T_FAST
    gidx_ref[...] = jnp.transpose(jnp.where(fast, (idx // 2) * NB + bidx, 0), (1, 0))
    par_ref[...] = (idx % 2).astype(jnp.float32)
    rare = (picked > 0.5) & jnp.logical_not(fast)
    rare_ref[...] = jnp.transpose(jnp.where(rare, idx * (DV * NB) + bidx, -1), (1, 0))
    total_ref[...] = cs[:, L - 1:L]


def _mlp_body(rows_ref, par_ref, total_ref, cnt_ref, w1t_ref, w1c_ref, b1_ref,
              w2t_ref, b2_ref, out_ref, *, DV, K):
    w2 = 2 * DV
    halfc = (lax.broadcasted_iota(jnp.int32, (1, w2), 1) >= DV).astype(jnp.float32)
    total = total_ref[...]
    lc = jnp.log1p(cnt_ref[...])
    h = lc * w1c_ref[...] + b1_ref[...]
    for j in range(K):
        xj = rows_ref[j]
        pj = par_ref[:, j:j + 1]
        keep = (total > float(j)).astype(jnp.float32)
        match = (1.0 - pj) * (1.0 - halfc) + pj * halfc
        xj = xj * (match * keep)
        h = h + jnp.dot(xj, w1t_ref[pl.ds(j * w2, w2), :],
                        preferred_element_type=jnp.float32)
    h = 0.5 * h * (1.0 + lax.erf(h * (2.0 ** -0.5)))
    out_ref[...] = jnp.dot(h, w2t_ref[...], preferred_element_type=jnp.float32) + b2_ref[...]


def _make_sc_gather(n_rows, d, dv, nb):
    info = plsc.get_sparse_core_info()
    nw = info.num_cores * info.num_subcores
    rows_per_w = n_rows // nw
    chunk = 128
    chunks = rows_per_w // chunk
    half = chunks // 2
    mesh = plsc.VectorSubcoreMesh(core_axis_name="c", subcore_axis_name="s")

    @functools.partial(
        pl.kernel, mesh=mesh,
        out_type=jax.ShapeDtypeStruct((n_rows, d), jnp.float32),
        compiler_params=pltpu.CompilerParams(
            use_tc_tiling_on_sc=True, needs_layout_passes=False),
        scratch_types=[
            pltpu.VMEM((chunks, chunk), jnp.int32),
            pltpu.VMEM((half * chunk, d), jnp.float32),
            pltpu.VMEM((chunks, chunk), jnp.int32),
            pltpu.VMEM((d, d), jnp.float32),
            pltpu.SemaphoreType.DMA,
        ],
    )
    def gather(idx_hbm, rare_hbm, table_hbm, vt2_hbm, out_hbm,
               idx_v, rows_v, rare_v, tile_v, sem):
        wid = lax.axis_index("s") * info.num_cores + lax.axis_index("c")
        pltpu.sync_copy(idx_hbm.at[pl.ds(wid * chunks, chunks)], idx_v)
        pltpu.sync_copy(rare_hbm.at[pl.ds(wid * chunks, chunks)], rare_v)
        lanes = lax.iota(jnp.int32, 16)

        def patch_slot(s_local, val):
            @pl.when(val >= 0)
            def _():
                l = val // (dv * nb)
                b = val - l * (dv * nb)
                q = l // 2
                colbase = (b // d) * d
                c = b - colbase
                pltpu.sync_copy(
                    vt2_hbm.at[pl.ds(q * d, d), pl.ds(colbase, d)], tile_v)
                for r8 in range(d // 16):
                    vals = plsc.load_gather(
                        tile_v, [lanes + 16 * r8, jnp.zeros((16,), jnp.int32) + c])
                    plsc.store_scatter(
                        rows_v,
                        [jnp.zeros((16,), jnp.int32) + s_local, lanes + 16 * r8],
                        vals)

        def patch_round(rnd):
            for r in range(half):
                def g2body(g2, carry, r=r):
                    gv = rare_v[rnd * half + r, pl.ds(g2 * 16, 16)]

                    @pl.when(jnp.max(gv) >= 0)
                    def _():
                        for i in range(16):
                            patch_slot(r * chunk + g2 * 16 + i, gv[i])
                    return carry
                lax.fori_loop(0, chunk // 16, g2body, 0)

        for rnd in range(2):
            copies = []
            for c in range(half):
                copies.append(pltpu.async_copy(
                    table_hbm.at[idx_v.at[rnd * half + c]],
                    rows_v.at[pl.ds(c * chunk, chunk)],
                    sem))
            for cp in copies:
                cp.wait()
            patch_round(rnd)
            pltpu.sync_copy(
                rows_v,
                out_hbm.at[pl.ds(wid * rows_per_w + rnd * half * chunk, half * chunk)])

    return gather


def kernel(v, batch_idx, mask, count, W1, b1, W2, b2):
    n_batch, L, dv = v.shape
    n_chains = mask.shape[0]
    hidden, d_in = W1.shape
    K = (d_in - 1) // dv
    qf = T_FAST // 2

    vt = jnp.transpose(v, (1, 2, 0))
    table3 = pl.pallas_call(
        _transpose_body,
        grid=(qf // 4,),
        in_specs=[pl.BlockSpec((8, dv, n_batch), lambda i: (i, 0, 0))],
        out_specs=pl.BlockSpec((4, n_batch, 2 * dv), lambda i: (i, 0, 0)),
        out_shape=jax.ShapeDtypeStruct((qf, n_batch, 2 * dv), jnp.float32),
    )(vt)
    table = table3.reshape(qf * n_batch, 2 * dv)
    vt2 = vt.reshape(L * dv, n_batch)

    maskf = mask.astype(jnp.float32)
    bidx = batch_idx.astype(jnp.int32).reshape(n_chains, 1)
    blk = 1024
    grid = n_chains // blk
    gidx, par, rare, total = pl.pallas_call(
        functools.partial(_select_body, L=L, K=K, NB=n_batch, DV=dv),
        grid=(grid,),
        in_specs=[
            pl.BlockSpec((blk, L), lambda i: (i, 0)),
            pl.BlockSpec((blk, 1), lambda i: (i, 0)),
        ],
        out_specs=[
            pl.BlockSpec((K, blk), lambda i: (0, i)),
            pl.BlockSpec((blk, K), lambda i: (i, 0)),
            pl.BlockSpec((K, blk), lambda i: (0, i)),
            pl.BlockSpec((blk, 1), lambda i: (i, 0)),
        ],
        out_shape=[
            jax.ShapeDtypeStruct((K, n_chains), jnp.int32),
            jax.ShapeDtypeStruct((n_chains, K), jnp.float32),
            jax.ShapeDtypeStruct((K, n_chains), jnp.int32),
            jax.ShapeDtypeStruct((n_chains, 1), jnp.float32),
        ],
    )(maskf, bidx)

    n_rows = n_chains * K
    nw = 32
    idx2d = gidx.reshape(n_rows // 128, 128)
    rare2d = rare.reshape(n_rows // 128, 128)
    rows = _make_sc_gather(n_rows, 2 * dv, dv, n_batch)(idx2d, rare2d, table, vt2)
    rows2 = rows.reshape(K, n_chains, 2 * dv)

    w1t = W1[:, :K * dv].T
    w1t_dup = jnp.broadcast_to(
        w1t.reshape(K, 1, dv, hidden), (K, 2, dv, hidden)).reshape(K * 2 * dv, hidden)
    w1c = W1[:, K * dv:].T
    w2t = W2.T
    out = pl.pallas_call(
        functools.partial(_mlp_body, DV=dv, K=K),
        grid=(grid,),
        in_specs=[
            pl.BlockSpec((K, blk, 2 * dv), lambda i: (0, i, 0)),
            pl.BlockSpec((blk, K), lambda i: (i, 0)),
            pl.BlockSpec((blk, 1), lambda i: (i, 0)),
            pl.BlockSpec((blk, 1), lambda i: (i, 0)),
            pl.BlockSpec((K * 2 * dv, hidden), lambda i: (0, 0)),
            pl.BlockSpec((1, hidden), lambda i: (0, 0)),
            pl.BlockSpec((1, hidden), lambda i: (0, 0)),
            pl.BlockSpec((hidden, dv), lambda i: (0, 0)),
            pl.BlockSpec((1, dv), lambda i: (0, 0)),
        ],
        out_specs=pl.BlockSpec((blk, dv), lambda i: (i, 0)),
        out_shape=jax.ShapeDtypeStruct((n_chains, dv), jnp.float32),
    )(rows2, par, total, count.reshape(n_chains, 1).astype(jnp.float32),
      w1t_dup, w1c, b1.reshape(1, hidden), w2t, b2.reshape(1, dv))
    return out
